# baseline (speedup 1.0000x reference)
import functools

import jax
import jax.numpy as jnp
from jax import lax
from jax.experimental import pallas as pl
from jax.experimental.pallas import tpu as pltpu
from jax.experimental.pallas import tpu_sc as plsc

LANES = 16
NUM_WORKERS = 32
IDX_CHUNK = 128


def _make_sc_kernel(W, B, V, D):
  b_per_w = B // NUM_WORKERS
  n_blocks = b_per_w // LANES
  copies = b_per_w // IDX_CHUNK
  mesh = plsc.VectorSubcoreMesh(core_axis_name="c", subcore_axis_name="s")

  @functools.partial(
      pl.kernel,
      out_type=(
          jax.ShapeDtypeStruct((W, D, B), jnp.float32),
          jax.ShapeDtypeStruct((W, B), jnp.float32),
          jax.ShapeDtypeStruct((B,), jnp.int32),
      ),
      mesh=mesh,
      compiler_params=pltpu.CompilerParams(
          needs_layout_passes=False, use_tc_tiling_on_sc=False),
      scratch_types=[
          pltpu.VMEM((W, b_per_w), jnp.int32),
          pltpu.VMEM((W, b_per_w), jnp.float32),
          pltpu.VMEM((b_per_w,), jnp.int32),
          pltpu.VMEM((b_per_w,), jnp.int32),
          pltpu.VMEM((W, b_per_w), jnp.int32),
          pltpu.VMEM((W, b_per_w), jnp.float32),
          pltpu.VMEM((W, b_per_w), jnp.float32),
          pltpu.VMEM((b_per_w, D), jnp.float32),
          pltpu.VMEM((D, b_per_w), jnp.float32),
          pltpu.SemaphoreType.DMA,
      ],
  )
  def sc_kernel(table_hbm, rew_hbm, idx_hbm, len_hbm,
                seq_hbm, mask_hbm, lens_hbm,
                idx_v, rew_v, len_v, lenc_v, gid_v, scale_v, mask_v,
                rows_v, panel_v, sem):
    wid = lax.axis_index("s") * 2 + lax.axis_index("c")
    b0 = wid * b_per_w

    pltpu.sync_copy(idx_hbm.at[:, pl.ds(b0, b_per_w)], idx_v)
    pltpu.sync_copy(rew_hbm.at[:, pl.ds(b0, b_per_w)], rew_v)
    pltpu.sync_copy(len_hbm.at[pl.ds(b0, b_per_w)], len_v)

    def blk_body(blk, carry):
      bi = blk * LANES + jnp.arange(LANES, dtype=jnp.int32)
      L = len_v[pl.ds(blk * LANES, LANES)]
      Lc = jnp.clip(L, 1, W)
      lenc_v[pl.ds(blk * LANES, LANES)] = jnp.clip(L, 0, W)
      for j in range(W):
        tj = jnp.where(j < Lc, Lc - 1 - j, j)
        g = plsc.load_gather(idx_v, [tj, bi])
        g = jnp.where(g == -1, V - 1, g)
        g = jnp.clip(g, 0, V - 1)
        r = plsc.load_gather(rew_v, [tj, bi])
        live = j < L
        m = jnp.where(live, jnp.float32(1.0), jnp.float32(0.0))
        s = jnp.minimum(r, jnp.float32(1.0)) * m
        gid_v[j, pl.ds(blk * LANES, LANES)] = g
        scale_v[j, pl.ds(blk * LANES, LANES)] = s
        mask_v[j, pl.ds(blk * LANES, LANES)] = m
      return carry

    lax.fori_loop(0, n_blocks, blk_body, 0)

    pltpu.sync_copy(mask_v, mask_hbm.at[:, pl.ds(b0, b_per_w)])
    pltpu.sync_copy(lenc_v, lens_hbm.at[pl.ds(b0, b_per_w)])

    def j_body(j, carry):
      cps = []
      for k in range(copies):
        cps.append(pltpu.async_copy(
            table_hbm.at[gid_v.at[j, pl.ds(k * IDX_CHUNK, IDX_CHUNK)]],
            rows_v.at[pl.ds(k * IDX_CHUNK, IDX_CHUNK)],
            sem))
      for cp in cps:
        cp.wait()

      def bg_body(bg, bcarry):
        bvec = bg * LANES + jnp.arange(LANES, dtype=jnp.int32)
        sv = scale_v[j, pl.ds(bg * LANES, LANES)]
        for d in range(D):
          vals = plsc.load_gather(
              rows_v, [bvec, jnp.full((LANES,), d, jnp.int32)])
          panel_v[d, pl.ds(bg * LANES, LANES)] = vals * sv
        return bcarry

      lax.fori_loop(0, n_blocks, bg_body, 0)
      pltpu.sync_copy(panel_v, seq_hbm.at[j, :, pl.ds(b0, b_per_w)])
      return carry

    lax.fori_loop(0, W, j_body, 0)

  return sc_kernel


def kernel(item_table, rewards, item_indices, lengths):
  W, B = item_indices.shape
  V, D = item_table.shape
  sc = _make_sc_kernel(W, B, V, D)
  seq_t, mask_t, len_states = sc(
      item_table, rewards, item_indices.astype(jnp.int32),
      lengths.astype(jnp.int32))
  seq = jnp.transpose(seq_t, (2, 0, 1))
  mask_bw = jnp.transpose(mask_t)[:, :, None]
  return seq, mask_bw, len_states
